# speedup vs baseline: 3.3132x; 3.3132x over previous
import jax
import jax.numpy as jnp
from jax.experimental import pallas as pl

_BS = 256


def _pe_add_kernel(x_ref, pe_ref, o_ref):
    o_ref[...] = x_ref[...] + pe_ref[...][None, :, :]


def kernel(x, pe_table):
    B, S, H = x.shape
    return pl.pallas_call(
        _pe_add_kernel,
        grid=(S // _BS,),
        in_specs=[
            pl.BlockSpec((B, _BS, H), lambda i: (0, i, 0)),
            pl.BlockSpec((_BS, H), lambda i: (i, 0)),
        ],
        out_specs=pl.BlockSpec((B, _BS, H), lambda i: (0, i, 0)),
        out_shape=jax.ShapeDtypeStruct((B, S, H), x.dtype),
    )(x, pe_table)
